# baseline (speedup 1.0000x reference)
import functools

import jax
import jax.numpy as jnp
from jax import lax
from jax.experimental import pallas as pl
from jax.experimental.pallas import tpu as pltpu
from jax.experimental.pallas import tpu_sc as plsc

N_NODES = 100000
OUT_DIM = 128
NE = 150000
PADN = 155648
ROWS_PER_ETYPE = PADN // OUT_DIM
EPT = PADN // 8
CH = 128
NCHUNK = EPT // CH
TOT_ROWS = 4 * ROWS_PER_ETYPE
REG = 0.01


def _sc_scores(table, src2d, dst2d, wmat):
    mesh = plsc.VectorSubcoreMesh(core_axis_name="c", subcore_axis_name="s")

    @functools.partial(
        pl.kernel,
        mesh=mesh,
        out_type=jax.ShapeDtypeStruct((TOT_ROWS, CH), jnp.float32),
        compiler_params=pltpu.CompilerParams(needs_layout_passes=False),
        scratch_types=[
            pltpu.VMEM((NCHUNK, CH), jnp.int32),
            pltpu.VMEM((NCHUNK, CH), jnp.int32),
            pltpu.VMEM((2, CH, OUT_DIM), jnp.float32),
            pltpu.VMEM((2, CH, OUT_DIM), jnp.float32),
            pltpu.VMEM((NCHUNK, CH), jnp.float32),
            pltpu.VMEM((OUT_DIM,), jnp.float32),
            pltpu.SemaphoreType.DMA,
            pltpu.SemaphoreType.DMA,
            pltpu.SemaphoreType.DMA,
            pltpu.SemaphoreType.DMA,
        ],
    )
    def k(table_hbm, src_hbm, dst_hbm, wmat_hbm, out_hbm,
          sidx, didx, srows, orows, scores, wrow,
          sem_s0, sem_o0, sem_s1, sem_o1):
        wid = lax.axis_index("s") * 2 + lax.axis_index("c")
        etype = wid // 8
        brow = wid * NCHUNK
        pltpu.sync_copy(wmat_hbm.at[etype], wrow)
        pltpu.sync_copy(src_hbm.at[pl.ds(brow, NCHUNK)], sidx)
        pltpu.sync_copy(dst_hbm.at[pl.ds(brow, NCHUNK)], didx)
        wv = [wrow[pl.ds(kk * 16, 16)] for kk in range(8)]
        last_lane = lax.iota(jnp.int32, 16) == 15
        sems = ((sem_s0, sem_o0), (sem_s1, sem_o1))

        def issue(g, b):
            pltpu.async_copy(table_hbm.at[sidx.at[g]], srows.at[b], sems[b][0])
            pltpu.async_copy(table_hbm.at[didx.at[g]], orows.at[b], sems[b][1])

        def wait(g, b):
            pltpu.make_async_copy(
                table_hbm.at[sidx.at[g]], srows.at[b], sems[b][0]).wait()
            pltpu.make_async_copy(
                table_hbm.at[didx.at[g]], orows.at[b], sems[b][1]).wait()

        issue(0, 0)
        issue(1, 1)

        def outer(gg, carry):
            for b in range(2):
                g = 2 * gg + b
                wait(g, b)
                rs = srows.at[b]
                ro = orows.at[b]
                gvec = jnp.full((16,), g, jnp.int32)

                def edge_body(e, c2):
                    ps = [(rs[e, pl.ds(kk * 16, 16)] * wv[kk])
                          * ro[e, pl.ds(kk * 16, 16)] for kk in range(8)]
                    t0 = (ps[0] + ps[1]) + (ps[2] + ps[3])
                    t1 = (ps[4] + ps[5]) + (ps[6] + ps[7])
                    tot = jnp.full((16,), jnp.sum(t0 + t1))
                    plsc.store_scatter(
                        scores, [gvec, jnp.full((16,), e, jnp.int32)],
                        tot, mask=last_lane)
                    return c2

                lax.fori_loop(0, CH, edge_body, 0, unroll=8)

                @pl.when(g + 2 < NCHUNK)
                def _():
                    issue(g + 2, b)
            return carry

        lax.fori_loop(0, NCHUNK // 2, outer, 0)
        pltpu.sync_copy(scores, out_hbm.at[pl.ds(brow, NCHUNK)])

    return k(table, src2d, dst2d, wmat)


def _tc_loss(scores4, labels4, embed, wmat):
    emb_blk = 4000
    n_blk = N_NODES // emb_blk

    def body(scores_ref, labels_ref, wmat_ref, embed_ref, out_ref):
        i = pl.program_id(0)

        @pl.when(i == 0)
        def _init():
            x = scores_ref[...]
            y = labels_ref[...]
            row = lax.broadcasted_iota(jnp.int32, x.shape, 0)
            col = lax.broadcasted_iota(jnp.int32, x.shape, 1)
            rin = row % ROWS_PER_ETYPE
            valid = (rin * OUT_DIM + col) < NE
            bce = jnp.maximum(x, 0.0) - x * y + jnp.log1p(jnp.exp(-jnp.abs(x)))
            bce = jnp.where(valid, bce, 0.0)
            w = wmat_ref[...]
            out_ref[0, 0] = jnp.sum(bce) / NE + REG * (jnp.sum(w * w) / OUT_DIM)

        blk = embed_ref[...]
        out_ref[0, 0] += REG * jnp.sum(blk * blk) / (N_NODES * OUT_DIM)

    out = pl.pallas_call(
        body,
        grid=(n_blk,),
        in_specs=[
            pl.BlockSpec((TOT_ROWS, OUT_DIM), lambda i: (0, 0)),
            pl.BlockSpec((TOT_ROWS, OUT_DIM), lambda i: (0, 0)),
            pl.BlockSpec((4, OUT_DIM), lambda i: (0, 0)),
            pl.BlockSpec((emb_blk, OUT_DIM), lambda i: (i, 0)),
        ],
        out_specs=pl.BlockSpec(memory_space=pltpu.SMEM),
        out_shape=jax.ShapeDtypeStruct((1, 1), jnp.float32),
    )(scores4, labels4, wmat, embed)
    return out[0, 0]


def kernel(embed_0,
           edges_rel0, edges_rel1, edges_rel2, edges_rel3,
           labels_rel0, labels_rel1, labels_rel2, labels_rel3,
           w_rel0, w_rel1, w_rel2, w_rel3):
    edges = [edges_rel0, edges_rel1, edges_rel2, edges_rel3]
    labels = [labels_rel0, labels_rel1, labels_rel2, labels_rel3]
    pad = PADN - NE
    src = jnp.concatenate([jnp.pad(ed[:, 0], (0, pad)) for ed in edges])
    dst = jnp.concatenate([jnp.pad(ed[:, 1], (0, pad)) for ed in edges])
    lab = jnp.concatenate([jnp.pad(lb, (0, pad)) for lb in labels])
    wmat = jnp.stack([w_rel0, w_rel1, w_rel2, w_rel3])

    scores = _sc_scores(embed_0, src.reshape(TOT_ROWS, CH),
                        dst.reshape(TOT_ROWS, CH), wmat)
    return _tc_loss(scores.reshape(TOT_ROWS, OUT_DIM),
                    lab.reshape(TOT_ROWS, OUT_DIM),
                    embed_0, wmat)


# speedup vs baseline: 1.0003x; 1.0003x over previous
import functools

import jax
import jax.numpy as jnp
from jax import lax
from jax.experimental import pallas as pl
from jax.experimental.pallas import tpu as pltpu
from jax.experimental.pallas import tpu_sc as plsc

N_NODES = 100000
OUT_DIM = 128
NE = 150000
PADN = 155648
ROWS_PER_ETYPE = PADN // OUT_DIM
EPT = PADN // 8
CH = 128
NCHUNK = EPT // CH
TOT_ROWS = 4 * ROWS_PER_ETYPE
REG = 0.01


def _sc_scores(table, src2d, dst2d, wmat):
    mesh = plsc.VectorSubcoreMesh(core_axis_name="c", subcore_axis_name="s")

    @functools.partial(
        pl.kernel,
        mesh=mesh,
        out_type=jax.ShapeDtypeStruct((TOT_ROWS, CH), jnp.float32),
        compiler_params=pltpu.CompilerParams(needs_layout_passes=False),
        scratch_types=[
            pltpu.VMEM((NCHUNK, CH), jnp.int32),
            pltpu.VMEM((NCHUNK, CH), jnp.int32),
            pltpu.VMEM((2, CH, OUT_DIM), jnp.float32),
            pltpu.VMEM((2, CH, OUT_DIM), jnp.float32),
            pltpu.VMEM((NCHUNK, CH), jnp.float32),
            pltpu.VMEM((OUT_DIM,), jnp.float32),
            pltpu.SemaphoreType.DMA,
            pltpu.SemaphoreType.DMA,
            pltpu.SemaphoreType.DMA,
            pltpu.SemaphoreType.DMA,
        ],
    )
    def k(table_hbm, src_hbm, dst_hbm, wmat_hbm, out_hbm,
          sidx, didx, srows, orows, scores, wrow,
          sem_s0, sem_o0, sem_s1, sem_o1):
        wid = lax.axis_index("s") * 2 + lax.axis_index("c")
        etype = wid // 8
        brow = wid * NCHUNK
        pltpu.sync_copy(wmat_hbm.at[etype], wrow)
        pltpu.sync_copy(src_hbm.at[pl.ds(brow, NCHUNK)], sidx)
        pltpu.sync_copy(dst_hbm.at[pl.ds(brow, NCHUNK)], didx)
        wv = [wrow[pl.ds(kk * 16, 16)] for kk in range(8)]
        last_lane = lax.iota(jnp.int32, 16) == 15
        sems = ((sem_s0, sem_o0), (sem_s1, sem_o1))

        def issue(g, b):
            pltpu.async_copy(table_hbm.at[sidx.at[g]], srows.at[b], sems[b][0])
            pltpu.async_copy(table_hbm.at[didx.at[g]], orows.at[b], sems[b][1])

        def wait(g, b):
            pltpu.make_async_copy(
                table_hbm.at[sidx.at[g]], srows.at[b], sems[b][0]).wait()
            pltpu.make_async_copy(
                table_hbm.at[didx.at[g]], orows.at[b], sems[b][1]).wait()

        issue(0, 0)
        issue(1, 1)

        def outer(gg, carry):
            for b in range(2):
                g = 2 * gg + b
                wait(g, b)
                rs = srows.at[b]
                ro = orows.at[b]
                gvec = jnp.full((16,), g, jnp.int32)

                def edge_body(e, c2):
                    ps = [(rs[e, pl.ds(kk * 16, 16)] * wv[kk])
                          * ro[e, pl.ds(kk * 16, 16)] for kk in range(8)]
                    t0 = (ps[0] + ps[1]) + (ps[2] + ps[3])
                    t1 = (ps[4] + ps[5]) + (ps[6] + ps[7])
                    tot = jnp.full((16,), jnp.sum(t0 + t1))
                    plsc.store_scatter(
                        scores, [gvec, jnp.full((16,), e, jnp.int32)],
                        tot, mask=last_lane)
                    return c2

                lax.fori_loop(0, CH, edge_body, 0)

                @pl.when(g + 2 < NCHUNK)
                def _():
                    issue(g + 2, b)
            return carry

        lax.fori_loop(0, NCHUNK // 2, outer, 0)
        pltpu.sync_copy(scores, out_hbm.at[pl.ds(brow, NCHUNK)])

    return k(table, src2d, dst2d, wmat)


def _tc_loss(scores4, labels4, embed, wmat):
    emb_blk = 4000
    n_blk = N_NODES // emb_blk

    def body(scores_ref, labels_ref, wmat_ref, embed_ref, out_ref):
        i = pl.program_id(0)

        @pl.when(i == 0)
        def _init():
            x = scores_ref[...]
            y = labels_ref[...]
            row = lax.broadcasted_iota(jnp.int32, x.shape, 0)
            col = lax.broadcasted_iota(jnp.int32, x.shape, 1)
            rin = row % ROWS_PER_ETYPE
            valid = (rin * OUT_DIM + col) < NE
            bce = jnp.maximum(x, 0.0) - x * y + jnp.log1p(jnp.exp(-jnp.abs(x)))
            bce = jnp.where(valid, bce, 0.0)
            w = wmat_ref[...]
            out_ref[0, 0] = jnp.sum(bce) / NE + REG * (jnp.sum(w * w) / OUT_DIM)

        blk = embed_ref[...]
        out_ref[0, 0] += REG * jnp.sum(blk * blk) / (N_NODES * OUT_DIM)

    out = pl.pallas_call(
        body,
        grid=(n_blk,),
        in_specs=[
            pl.BlockSpec((TOT_ROWS, OUT_DIM), lambda i: (0, 0)),
            pl.BlockSpec((TOT_ROWS, OUT_DIM), lambda i: (0, 0)),
            pl.BlockSpec((4, OUT_DIM), lambda i: (0, 0)),
            pl.BlockSpec((emb_blk, OUT_DIM), lambda i: (i, 0)),
        ],
        out_specs=pl.BlockSpec(memory_space=pltpu.SMEM),
        out_shape=jax.ShapeDtypeStruct((1, 1), jnp.float32),
    )(scores4, labels4, wmat, embed)
    return out[0, 0]


def kernel(embed_0,
           edges_rel0, edges_rel1, edges_rel2, edges_rel3,
           labels_rel0, labels_rel1, labels_rel2, labels_rel3,
           w_rel0, w_rel1, w_rel2, w_rel3):
    edges = [edges_rel0, edges_rel1, edges_rel2, edges_rel3]
    labels = [labels_rel0, labels_rel1, labels_rel2, labels_rel3]
    pad = PADN - NE
    src = jnp.concatenate([jnp.pad(ed[:, 0], (0, pad)) for ed in edges])
    dst = jnp.concatenate([jnp.pad(ed[:, 1], (0, pad)) for ed in edges])
    lab = jnp.concatenate([jnp.pad(lb, (0, pad)) for lb in labels])
    wmat = jnp.stack([w_rel0, w_rel1, w_rel2, w_rel3])

    scores = _sc_scores(embed_0, src.reshape(TOT_ROWS, CH),
                        dst.reshape(TOT_ROWS, CH), wmat)
    return _tc_loss(scores.reshape(TOT_ROWS, OUT_DIM),
                    lab.reshape(TOT_ROWS, OUT_DIM),
                    embed_0, wmat)
